# baseline (speedup 1.0000x reference)
import functools

import jax
import jax.numpy as jnp
from jax import lax
from jax.experimental import pallas as pl
from jax.experimental.pallas import tpu as pltpu
from jax.experimental.pallas import tpu_sc as plsc

N_ROWS = 320000
D = 128
N_WORKERS = 32
ROWS_PER_W = N_ROWS // N_WORKERS
CHUNK = 200
N_CHUNKS = ROWS_PER_W // CHUNK
NV = D // 16


def _sc_body(feat_hbm, lbl_hbm, edge_out, meta_out,
             lbl_v, fbuf, ebuf, mbuf, intbuf, sem0, sem1):
    c = lax.axis_index("c")
    s = lax.axis_index("s")
    wid = s * 2 + c
    row0 = wid * ROWS_PER_W

    pltpu.sync_copy(lbl_hbm.at[pl.ds(row0, ROWS_PER_W)],
                    lbl_v.at[pl.ds(0, ROWS_PER_W)])

    sems = (sem0, sem1)

    def start(ci, b):
        pltpu.async_copy(feat_hbm.at[pl.ds(row0 + ci * CHUNK, CHUNK)],
                         fbuf.at[b], sems[b])

    def wait(ci, b):
        pltpu.make_async_copy(feat_hbm.at[pl.ds(row0 + ci * CHUNK, CHUNK)],
                              fbuf.at[b], sems[b]).wait()

    start(0, 0)
    start(1, 1)

    zero_v = jnp.zeros((16,), jnp.float32)
    one_v = jnp.ones((16,), jnp.float32)
    ii = lax.iota(jnp.int32, 16)
    intbuf[pl.ds(0, 16)] = zero_v

    init = dict(
        prev=jnp.int32(-1),
        cnt=jnp.float32(0.0),
        nseg=jnp.int32(0),
        acc=[zero_v] * NV,
        sqv=zero_v,
    )

    def row_step(b, gr0, r, st):
        gr = gr0 + r
        lbl = lbl_v[pl.ds(gr, 16)][0]
        boundary = jnp.logical_and(gr > 0, lbl != st["prev"])
        acc = st["acc"]

        @pl.when(jnp.logical_and(boundary, st["nseg"] == 0))
        def _first():
            for j in range(NV):
                ebuf[pl.ds(j * 16, 16)] = acc[j]
            flbl = st["prev"].astype(jnp.float32)
            mbuf[pl.ds(0, 16)] = (
                jnp.where(ii == 0, flbl, zero_v)
                + jnp.where(ii == 1, st["cnt"], zero_v))

        @pl.when(jnp.logical_and(boundary, st["nseg"] != 0))
        def _interior():
            n2v = acc[0] * acc[0]
            for j in range(1, NV):
                n2v = n2v + acc[j] * acc[j]
            cntv = zero_v + st["cnt"]
            intbuf[pl.ds(0, 16)] = (intbuf[pl.ds(0, 16)]
                                    + n2v * (one_v / cntv))

        nseg = jnp.where(boundary, st["nseg"] + 1, st["nseg"])
        cnt = jnp.where(boundary, jnp.float32(0.0), st["cnt"]) + 1.0

        sqv = st["sqv"]
        new_acc = []
        for j in range(NV):
            v = fbuf[b, r, pl.ds(j * 16, 16)]
            a = jnp.where(boundary, zero_v, acc[j])
            new_acc.append(a + v)
            sqv = sqv + v * v
        return dict(prev=lbl, cnt=cnt, nseg=nseg, acc=new_acc, sqv=sqv)

    def chunk_pair(k, st):
        for b in range(2):
            ci = 2 * k + b
            wait(ci, b)
            st = lax.fori_loop(
                0, CHUNK,
                functools.partial(row_step, b, ci * CHUNK),
                st)

            @pl.when(ci + 2 < N_CHUNKS)
            def _():
                start(ci + 2, b)
        return st

    st = lax.fori_loop(0, N_CHUNKS // 2, chunk_pair, init)

    last_lbl = st["prev"].astype(jnp.float32)

    @pl.when(st["nseg"] == 0)
    def _lone():
        for j in range(NV):
            ebuf[pl.ds(j * 16, 16)] = zero_v
        mbuf[pl.ds(0, 16)] = jnp.where(ii == 0, last_lbl, zero_v)

    for j in range(NV):
        ebuf[pl.ds(D + j * 16, 16)] = st["acc"][j]
    mlast = (jnp.where(ii == 0, last_lbl, zero_v)
             + jnp.where(ii == 1, st["cnt"], zero_v))
    mbuf[pl.ds(16, 16)] = mlast
    mbuf[pl.ds(32, 16)] = st["sqv"]
    mbuf[pl.ds(48, 16)] = intbuf[pl.ds(0, 16)]

    pltpu.sync_copy(ebuf, edge_out.at[pl.ds(wid * 2 * D, 2 * D)])
    pltpu.sync_copy(mbuf.at[pl.ds(0, 32)],
                    meta_out.at[pl.ds(wid * 32, 32)])
    pltpu.sync_copy(mbuf.at[pl.ds(32, 16)],
                    meta_out.at[pl.ds((2 * N_WORKERS + wid) * 16, 16)])
    pltpu.sync_copy(mbuf.at[pl.ds(48, 16)],
                    meta_out.at[pl.ds((3 * N_WORKERS + wid) * 16, 16)])


_sc_pass = pl.kernel(
    _sc_body,
    out_type=[
        jax.ShapeDtypeStruct((2 * N_WORKERS * D,), jnp.float32),
        jax.ShapeDtypeStruct((4 * N_WORKERS * 16,), jnp.float32),
    ],
    mesh=plsc.VectorSubcoreMesh(core_axis_name="c", subcore_axis_name="s"),
    scratch_types=[
        pltpu.VMEM((ROWS_PER_W + 16,), jnp.int32),
        pltpu.VMEM((2, CHUNK, D), jnp.float32),
        pltpu.VMEM((2 * D,), jnp.float32),
        pltpu.VMEM((4 * 16,), jnp.float32),
        pltpu.VMEM((16,), jnp.float32),
        pltpu.SemaphoreType.DMA,
        pltpu.SemaphoreType.DMA,
    ],
)


def _tc_combine_body(edge_ref, meta_ref, out_ref):
    ne = 2 * N_WORKERS
    e = edge_ref[...]
    m = meta_ref[...]
    lbl = m[:ne, 0:1]
    cnt = m[:ne, 1:2]
    sx = jnp.sum(m[ne:ne + N_WORKERS, :])
    interior = jnp.sum(m[ne + N_WORKERS:, :])
    lbl_row = lbl.reshape(1, ne)
    same = (lbl == lbl_row).astype(jnp.float32)
    gsum = jnp.dot(same, e, preferred_element_type=jnp.float32)
    gcnt = jnp.dot(same, cnt, preferred_element_type=jnp.float32)
    ir = lax.broadcasted_iota(jnp.int32, (ne, ne), 0)
    ic = lax.broadcasted_iota(jnp.int32, (ne, ne), 1)
    before = jnp.sum(same * (ic < ir).astype(jnp.float32), axis=1,
                     keepdims=True)
    first = (before == 0.0).astype(jnp.float32)
    gn2 = jnp.sum(gsum * gsum, axis=1, keepdims=True)
    contrib = jnp.sum(first * gn2 / jnp.maximum(gcnt, 1.0))
    out_ref[0, 0] = (sx - (interior + contrib)) / jnp.float32(N_ROWS * D)


_tc_combine = pl.pallas_call(
    _tc_combine_body,
    out_shape=jax.ShapeDtypeStruct((1, 1), jnp.float32),
    out_specs=pl.BlockSpec(memory_space=pltpu.SMEM),
)


def kernel(s_feature, s_labels):
    labels = s_labels.astype(jnp.int32)
    edge, meta = _sc_pass(s_feature, labels)
    edge = edge.reshape(2 * N_WORKERS, D)
    meta = meta.reshape(4 * N_WORKERS, 16)
    out = _tc_combine(edge, meta)
    return out[0, 0]


# speedup vs baseline: 1.0898x; 1.0898x over previous
import functools

import jax
import jax.numpy as jnp
from jax import lax
from jax.experimental import pallas as pl
from jax.experimental.pallas import tpu as pltpu
from jax.experimental.pallas import tpu_sc as plsc

N_ROWS = 320000
D = 128
N_WORKERS = 32
ROWS_PER_W = N_ROWS // N_WORKERS
CHUNK = 200
N_CHUNKS = ROWS_PER_W // CHUNK
NV = D // 16


def _sc_body(feat_hbm, lbl_hbm, edge_out, meta_out,
             lbl_v, fbuf, ebuf, mbuf, febuf, fmbuf, intbuf, sem0, sem1):
    c = lax.axis_index("c")
    s = lax.axis_index("s")
    wid = s * 2 + c
    row0 = wid * ROWS_PER_W

    pltpu.sync_copy(lbl_hbm.at[pl.ds(row0, ROWS_PER_W)],
                    lbl_v.at[pl.ds(0, ROWS_PER_W)])

    sems = (sem0, sem1)

    def start(ci, b):
        pltpu.async_copy(feat_hbm.at[pl.ds(row0 + ci * CHUNK, CHUNK)],
                         fbuf.at[b], sems[b])

    def wait(ci, b):
        pltpu.make_async_copy(feat_hbm.at[pl.ds(row0 + ci * CHUNK, CHUNK)],
                              fbuf.at[b], sems[b]).wait()

    start(0, 0)
    start(1, 1)

    zero_v = jnp.zeros((16,), jnp.float32)
    one_v = jnp.ones((16,), jnp.float32)
    ii = lax.iota(jnp.int32, 16)

    for j in range(NV):
        febuf[pl.ds(j * 16, 16)] = zero_v
    fmbuf[pl.ds(0, 16)] = zero_v
    intbuf[pl.ds(0, 16)] = zero_v

    init = dict(
        prev=lbl_v[pl.ds(0, 16)][0],
        cnt=jnp.float32(0.0),
        nseg=jnp.int32(0),
        acc=[zero_v] * NV,
        sq=[zero_v] * NV,
    )

    def row_step(b, gr0, r, st):
        gr = gr0 + r
        lbl = lbl_v[pl.ds(gr, 16)][0]
        boundary = lbl != st["prev"]
        acc = st["acc"]
        cnt0 = st["cnt"]

        @pl.when(jnp.logical_and(boundary, st["nseg"] == 0))
        def _first():
            for j in range(NV):
                febuf[pl.ds(j * 16, 16)] = acc[j]
            flbl = st["prev"].astype(jnp.float32)
            fmbuf[pl.ds(0, 16)] = (jnp.where(ii == 0, flbl, zero_v)
                                   + jnp.where(ii == 1, cnt0, zero_v))

        @pl.when(jnp.logical_and(boundary, st["nseg"] != 0))
        def _interior():
            n2v = acc[0] * acc[0]
            for j in range(1, NV):
                n2v = n2v + acc[j] * acc[j]
            cntv = zero_v + cnt0
            intbuf[pl.ds(0, 16)] = (intbuf[pl.ds(0, 16)]
                                    + n2v * (one_v / cntv))

        nseg = jnp.where(boundary, st["nseg"] + 1, st["nseg"])
        cnt = jnp.where(boundary, jnp.float32(0.0), cnt0) + 1.0

        new_acc = []
        new_sq = []
        for j in range(NV):
            v = fbuf[b, r, pl.ds(j * 16, 16)]
            a = jnp.where(boundary, zero_v, acc[j])
            new_acc.append(a + v)
            new_sq.append(st["sq"][j] + v * v)
        return dict(prev=lbl, cnt=cnt, nseg=nseg, acc=new_acc, sq=new_sq)

    def chunk_pair(k, st):
        for b in range(2):
            ci = 2 * k + b
            wait(ci, b)
            st = lax.fori_loop(
                0, CHUNK,
                functools.partial(row_step, b, ci * CHUNK),
                st)

            @pl.when(ci + 2 < N_CHUNKS)
            def _():
                start(ci + 2, b)
        return st

    st = lax.fori_loop(0, N_CHUNKS // 2, chunk_pair, init)

    last_lbl = st["prev"].astype(jnp.float32)
    sqv = st["sq"][0]
    for j in range(1, NV):
        sqv = sqv + st["sq"][j]
    for j in range(NV):
        ebuf[0, pl.ds(j * 16, 16)] = febuf[pl.ds(j * 16, 16)]
        ebuf[1, pl.ds(j * 16, 16)] = st["acc"][j]

    mlast = (jnp.where(ii == 0, last_lbl, zero_v)
             + jnp.where(ii == 1, st["cnt"], zero_v))
    mbuf[0, pl.ds(0, 16)] = fmbuf[pl.ds(0, 16)]
    mbuf[1, pl.ds(0, 16)] = mlast
    mbuf[2, pl.ds(0, 16)] = sqv
    mbuf[3, pl.ds(0, 16)] = intbuf[pl.ds(0, 16)]

    pltpu.sync_copy(ebuf, edge_out.at[pl.ds(2 * wid, 2)])
    pltpu.sync_copy(mbuf.at[pl.ds(0, 2)], meta_out.at[pl.ds(2 * wid, 2)])
    pltpu.sync_copy(mbuf.at[pl.ds(2, 1)], meta_out.at[pl.ds(2 * N_WORKERS + wid, 1)])
    pltpu.sync_copy(mbuf.at[pl.ds(3, 1)],
                    meta_out.at[pl.ds(3 * N_WORKERS + wid, 1)])


_sc_pass = pl.kernel(
    _sc_body,
    out_type=[
        jax.ShapeDtypeStruct((2 * N_WORKERS, D), jnp.float32),
        jax.ShapeDtypeStruct((4 * N_WORKERS, 16), jnp.float32),
    ],
    mesh=plsc.VectorSubcoreMesh(core_axis_name="c", subcore_axis_name="s"),
    scratch_types=[
        pltpu.VMEM((ROWS_PER_W + 16,), jnp.int32),
        pltpu.VMEM((2, CHUNK, D), jnp.float32),
        pltpu.VMEM((2, D), jnp.float32),
        pltpu.VMEM((4, 16), jnp.float32),
        pltpu.VMEM((D,), jnp.float32),
        pltpu.VMEM((16,), jnp.float32),
        pltpu.VMEM((16,), jnp.float32),
        pltpu.SemaphoreType.DMA,
        pltpu.SemaphoreType.DMA,
    ],
)


def _tc_combine_body(edge_ref, meta_ref, out_ref):
    ne = 2 * N_WORKERS
    e = edge_ref[...]
    m = meta_ref[...]
    lbl = m[:ne, 0:1]
    cnt = m[:ne, 1:2]
    sx = jnp.sum(m[ne:ne + N_WORKERS, :])
    interior = jnp.sum(m[ne + N_WORKERS:, :])
    lbl_row = lbl.reshape(1, ne)
    same = (lbl == lbl_row).astype(jnp.float32)
    gsum = jnp.dot(same, e, preferred_element_type=jnp.float32)
    gcnt = jnp.dot(same, cnt, preferred_element_type=jnp.float32)
    ir = lax.broadcasted_iota(jnp.int32, (ne, ne), 0)
    ic = lax.broadcasted_iota(jnp.int32, (ne, ne), 1)
    before = jnp.sum(same * (ic < ir).astype(jnp.float32), axis=1,
                     keepdims=True)
    first = (before == 0.0).astype(jnp.float32)
    gn2 = jnp.sum(gsum * gsum, axis=1, keepdims=True)
    contrib = jnp.sum(first * gn2 / jnp.maximum(gcnt, 1.0))
    out_ref[0, 0] = (sx - (interior + contrib)) / jnp.float32(N_ROWS * D)


_tc_combine = pl.pallas_call(
    _tc_combine_body,
    out_shape=jax.ShapeDtypeStruct((1, 1), jnp.float32),
    out_specs=pl.BlockSpec(memory_space=pltpu.SMEM),
)


def kernel(s_feature, s_labels):
    labels = s_labels.astype(jnp.int32)
    edge, meta = _sc_pass(s_feature, labels)
    out = _tc_combine(edge, meta)
    return out[0, 0]


# speedup vs baseline: 2.3827x; 2.1864x over previous
import functools

import jax
import jax.numpy as jnp
from jax import lax
from jax.experimental import pallas as pl
from jax.experimental.pallas import tpu as pltpu
from jax.experimental.pallas import tpu_sc as plsc

N_ROWS = 320000
D = 128
N_WORKERS = 32
ROWS_PER_W = N_ROWS // N_WORKERS
CHUNK = 200
N_CHUNKS = ROWS_PER_W // CHUNK
NV = D // 16
B = 8
N_BLK = CHUNK // B


def _sc_body(feat_hbm, lbl_hbm, edge_out, meta_out,
             lbl_v, fbuf, ebuf, mbuf, febuf, fmeta, intbuf, scal, corrb,
             sem0, sem1):
    c = lax.axis_index("c")
    s = lax.axis_index("s")
    wid = s * 2 + c
    row0 = wid * ROWS_PER_W

    pltpu.sync_copy(lbl_hbm.at[pl.ds(row0, ROWS_PER_W)],
                    lbl_v.at[pl.ds(0, ROWS_PER_W)])

    sems = (sem0, sem1)

    def start(ci, b):
        pltpu.async_copy(feat_hbm.at[pl.ds(row0 + ci * CHUNK, CHUNK)],
                         fbuf.at[b], sems[b])

    def wait(ci, b):
        pltpu.make_async_copy(feat_hbm.at[pl.ds(row0 + ci * CHUNK, CHUNK)],
                              fbuf.at[b], sems[b]).wait()

    start(0, 0)
    start(1, 1)

    zero_v = jnp.zeros((16,), jnp.float32)
    one_v = jnp.ones((16,), jnp.float32)
    ii = lax.iota(jnp.int32, 16)

    for j in range(NV):
        febuf[pl.ds(j * 16, 16)] = zero_v
    fmeta[pl.ds(0, 16)] = zero_v
    intbuf[pl.ds(0, 16)] = zero_v
    scal[pl.ds(0, 16)] = zero_v

    init = dict(
        prev=lbl_v[pl.ds(0, 16)][0],
        cnt=jnp.float32(0.0),
        acc=[zero_v] * NV,
        sq=[zero_v] * NV,
    )

    def blk_step(b, gr0c, blk, st):
        gr0 = gr0c + blk * B
        prev = st["prev"]
        lbl_last = lbl_v[pl.ds(gr0 + B - 1, 16)][0]
        hasb = lbl_last != prev

        sq = st["sq"]
        tj = [zero_v] * NV
        new_sq = []
        for j in range(NV):
            sqj = sq[j]
            t = tj[j]
            for rr in range(B):
                v = fbuf[b, blk * B + rr, pl.ds(j * 16, 16)]
                sqj = sqj + v * v
                t = t + v
            new_sq.append(sqj)
            tj[j] = t

        acc = st["acc"]

        @pl.when(hasb)
        def _fixup():
            a = list(acc)
            cc = st["cnt"]
            p = prev
            nsegf = scal[pl.ds(0, 16)][0]
            fe = [febuf[pl.ds(j * 16, 16)] for j in range(NV)]
            fm = fmeta[pl.ds(0, 16)]
            intv = intbuf[pl.ds(0, 16)]
            for rr in range(B):
                lblr = lbl_v[pl.ds(gr0 + rr, 16)][0]
                bnd = lblr != p
                isf = nsegf == 0.0
                bf = jnp.logical_and(bnd, isf)
                bi = jnp.logical_and(bnd, jnp.logical_not(isf))
                n2 = a[0] * a[0]
                for j in range(1, NV):
                    n2 = n2 + a[j] * a[j]
                cv = zero_v + cc
                w = jnp.where(bi, one_v / jnp.maximum(cv, one_v), zero_v)
                intv = intv + n2 * w
                pf = p.astype(jnp.float32)
                fmn = (jnp.where(ii == 0, pf, zero_v)
                       + jnp.where(ii == 1, cc, zero_v))
                fm = jnp.where(bf, fmn, fm)
                fe = [jnp.where(bf, a[j], fe[j]) for j in range(NV)]
                nsegf = jnp.where(bnd, nsegf + 1.0, nsegf)
                cc = jnp.where(bnd, jnp.float32(0.0), cc) + 1.0
                for j in range(NV):
                    v = fbuf[b, blk * B + rr, pl.ds(j * 16, 16)]
                    a[j] = jnp.where(bnd, zero_v, a[j]) + v
                p = lblr
            for j in range(NV):
                corrb[pl.ds(j * 16, 16)] = a[j]
                febuf[pl.ds(j * 16, 16)] = fe[j]
            corrb[pl.ds(NV * 16, 16)] = zero_v + cc
            fmeta[pl.ds(0, 16)] = fm
            intbuf[pl.ds(0, 16)] = intv
            scal[pl.ds(0, 16)] = zero_v + nsegf

        new_acc = [
            jnp.where(hasb, corrb[pl.ds(j * 16, 16)], acc[j] + tj[j])
            for j in range(NV)
        ]
        cnt = jnp.where(hasb, corrb[pl.ds(NV * 16, 16)][0],
                        st["cnt"] + jnp.float32(B))
        return dict(prev=lbl_last, cnt=cnt, acc=new_acc, sq=new_sq)

    def chunk_pair(k, st):
        for b in range(2):
            ci = 2 * k + b
            wait(ci, b)
            st = lax.fori_loop(
                0, N_BLK,
                functools.partial(blk_step, b, ci * CHUNK),
                st)

            @pl.when(ci + 2 < N_CHUNKS)
            def _():
                start(ci + 2, b)
        return st

    st = lax.fori_loop(0, N_CHUNKS // 2, chunk_pair, init)

    last_lbl = st["prev"].astype(jnp.float32)
    sqv = st["sq"][0]
    for j in range(1, NV):
        sqv = sqv + st["sq"][j]
    for j in range(NV):
        ebuf[0, pl.ds(j * 16, 16)] = febuf[pl.ds(j * 16, 16)]
        ebuf[1, pl.ds(j * 16, 16)] = st["acc"][j]

    mlast = (jnp.where(ii == 0, last_lbl, zero_v)
             + jnp.where(ii == 1, st["cnt"], zero_v))
    mbuf[0, pl.ds(0, 16)] = fmeta[pl.ds(0, 16)]
    mbuf[1, pl.ds(0, 16)] = mlast
    mbuf[2, pl.ds(0, 16)] = sqv
    mbuf[3, pl.ds(0, 16)] = intbuf[pl.ds(0, 16)]

    pltpu.sync_copy(ebuf, edge_out.at[pl.ds(2 * wid, 2)])
    pltpu.sync_copy(mbuf.at[pl.ds(0, 2)], meta_out.at[pl.ds(2 * wid, 2)])
    pltpu.sync_copy(mbuf.at[pl.ds(2, 1)], meta_out.at[pl.ds(2 * N_WORKERS + wid, 1)])
    pltpu.sync_copy(mbuf.at[pl.ds(3, 1)],
                    meta_out.at[pl.ds(3 * N_WORKERS + wid, 1)])


_sc_pass = pl.kernel(
    _sc_body,
    out_type=[
        jax.ShapeDtypeStruct((2 * N_WORKERS, D), jnp.float32),
        jax.ShapeDtypeStruct((4 * N_WORKERS, 16), jnp.float32),
    ],
    mesh=plsc.VectorSubcoreMesh(core_axis_name="c", subcore_axis_name="s"),
    scratch_types=[
        pltpu.VMEM((ROWS_PER_W + 16,), jnp.int32),
        pltpu.VMEM((2, CHUNK, D), jnp.float32),
        pltpu.VMEM((2, D), jnp.float32),
        pltpu.VMEM((4, 16), jnp.float32),
        pltpu.VMEM((D,), jnp.float32),
        pltpu.VMEM((16,), jnp.float32),
        pltpu.VMEM((16,), jnp.float32),
        pltpu.VMEM((16,), jnp.float32),
        pltpu.VMEM((D + 16,), jnp.float32),
        pltpu.SemaphoreType.DMA,
        pltpu.SemaphoreType.DMA,
    ],
)


def _tc_combine_body(edge_ref, meta_ref, out_ref):
    ne = 2 * N_WORKERS
    e = edge_ref[...]
    m = meta_ref[...]
    lbl = m[:ne, 0:1]
    cnt = m[:ne, 1:2]
    sx = jnp.sum(m[ne:ne + N_WORKERS, :])
    interior = jnp.sum(m[ne + N_WORKERS:, :])
    lbl_row = lbl.reshape(1, ne)
    same = (lbl == lbl_row).astype(jnp.float32)
    gsum = jnp.dot(same, e, preferred_element_type=jnp.float32)
    gcnt = jnp.dot(same, cnt, preferred_element_type=jnp.float32)
    ir = lax.broadcasted_iota(jnp.int32, (ne, ne), 0)
    ic = lax.broadcasted_iota(jnp.int32, (ne, ne), 1)
    before = jnp.sum(same * (ic < ir).astype(jnp.float32), axis=1,
                     keepdims=True)
    first = (before == 0.0).astype(jnp.float32)
    gn2 = jnp.sum(gsum * gsum, axis=1, keepdims=True)
    contrib = jnp.sum(first * gn2 / jnp.maximum(gcnt, 1.0))
    out_ref[0, 0] = (sx - (interior + contrib)) / jnp.float32(N_ROWS * D)


_tc_combine = pl.pallas_call(
    _tc_combine_body,
    out_shape=jax.ShapeDtypeStruct((1, 1), jnp.float32),
    out_specs=pl.BlockSpec(memory_space=pltpu.SMEM),
)


def kernel(s_feature, s_labels):
    labels = s_labels.astype(jnp.int32)
    edge, meta = _sc_pass(s_feature, labels)
    out = _tc_combine(edge, meta)
    return out[0, 0]


# speedup vs baseline: 3.1420x; 1.3186x over previous
import functools

import jax
import jax.numpy as jnp
from jax import lax
from jax.experimental import pallas as pl
from jax.experimental.pallas import tpu as pltpu
from jax.experimental.pallas import tpu_sc as plsc

N_ROWS = 320000
D = 128
N_WORKERS = 32
ROWS_PER_W = N_ROWS // N_WORKERS
CHUNK = 200
N_CHUNKS = ROWS_PER_W // CHUNK
NV = D // 16


def _sc_body(feat_hbm, lbl_hbm, edge_out, meta_out,
             lbl_v, fbuf, ebuf, mbuf, sem0, sem1):
    c = lax.axis_index("c")
    s = lax.axis_index("s")
    wid = s * 2 + c
    row0 = wid * ROWS_PER_W

    pltpu.sync_copy(lbl_hbm.at[pl.ds(row0, ROWS_PER_W)],
                    lbl_v.at[pl.ds(0, ROWS_PER_W)])

    sems = (sem0, sem1)

    def start(ci, b):
        pltpu.async_copy(feat_hbm.at[pl.ds(row0 + ci * CHUNK, CHUNK)],
                         fbuf.at[b], sems[b])

    def wait(ci, b):
        pltpu.make_async_copy(feat_hbm.at[pl.ds(row0 + ci * CHUNK, CHUNK)],
                              fbuf.at[b], sems[b]).wait()

    start(0, 0)
    start(1, 1)

    zero_v = jnp.zeros((16,), jnp.float32)
    init = dict(
        prev=jnp.int32(-1),
        cnt=jnp.float32(0.0),
        nseg=jnp.int32(0),
        interior=zero_v,
        flbl=jnp.float32(0.0),
        fcnt=jnp.float32(0.0),
        acc=[zero_v] * NV,
        sq=[zero_v] * NV,
        facc=[zero_v] * NV,
    )

    def row_step(b, gr0, r, st):
        gr = gr0 + r
        lbl = lbl_v[pl.ds(gr, 16)][0]
        boundary = jnp.logical_and(gr > 0, lbl != st["prev"])
        is_first = st["nseg"] == 0
        bf = jnp.logical_and(boundary, is_first)
        bi = jnp.logical_and(boundary, jnp.logical_not(is_first))

        acc = st["acc"]
        n2v = acc[0] * acc[0]
        for j in range(1, NV):
            n2v = n2v + acc[j] * acc[j]
        cntv = zero_v + st["cnt"]
        wv = jnp.where(bi, 1.0 / jnp.maximum(cntv, 1.0), zero_v)
        interior = st["interior"] + n2v * wv

        flbl = jnp.where(bf, st["prev"].astype(jnp.float32), st["flbl"])
        fcnt = jnp.where(bf, st["cnt"], st["fcnt"])
        facc = [jnp.where(bf, acc[j], st["facc"][j]) for j in range(NV)]
        nseg = jnp.where(boundary, st["nseg"] + 1, st["nseg"])
        cnt = jnp.where(boundary, jnp.float32(0.0), st["cnt"]) + 1.0

        sq = st["sq"]
        new_acc = []
        new_sq = []
        for j in range(NV):
            v = fbuf[b, r, pl.ds(j * 16, 16)]
            a = jnp.where(boundary, zero_v, acc[j])
            new_acc.append(a + v)
            new_sq.append(sq[j] + v * v)
        return dict(prev=lbl, cnt=cnt, nseg=nseg, interior=interior,
                    flbl=flbl, fcnt=fcnt, acc=new_acc, sq=new_sq, facc=facc)

    def row_pair(b, gr0, r, st):
        st = row_step(b, gr0, 2 * r, st)
        return row_step(b, gr0, 2 * r + 1, st)

    def chunk_pair(k, st):
        for b in range(2):
            ci = 2 * k + b
            wait(ci, b)
            st = lax.fori_loop(
                0, CHUNK // 2,
                functools.partial(row_pair, b, ci * CHUNK),
                st)

            @pl.when(ci + 2 < N_CHUNKS)
            def _():
                start(ci + 2, b)
        return st

    st = lax.fori_loop(0, N_CHUNKS // 2, chunk_pair, init)

    nseg = st["nseg"]
    last_lbl = st["prev"].astype(jnp.float32)
    flbl = jnp.where(nseg == 0, last_lbl, st["flbl"])
    fcnt = st["fcnt"]
    sqv = st["sq"][0]
    for j in range(1, NV):
        sqv = sqv + st["sq"][j]
    for j in range(NV):
        ebuf[0, pl.ds(j * 16, 16)] = st["facc"][j]
        ebuf[1, pl.ds(j * 16, 16)] = st["acc"][j]

    ii = lax.iota(jnp.int32, 16)
    zf = jnp.zeros((16,), jnp.float32)
    mfirst = jnp.where(ii == 0, flbl, zf) + jnp.where(ii == 1, fcnt, zf)
    mlast = (jnp.where(ii == 0, last_lbl, zf)
             + jnp.where(ii == 1, st["cnt"], zf))
    mbuf[0, pl.ds(0, 16)] = mfirst
    mbuf[1, pl.ds(0, 16)] = mlast
    mbuf[2, pl.ds(0, 16)] = sqv
    mbuf[3, pl.ds(0, 16)] = st["interior"]

    pltpu.sync_copy(ebuf, edge_out.at[pl.ds(2 * wid, 2)])
    pltpu.sync_copy(mbuf.at[pl.ds(0, 2)], meta_out.at[pl.ds(2 * wid, 2)])
    pltpu.sync_copy(mbuf.at[pl.ds(2, 1)], meta_out.at[pl.ds(2 * N_WORKERS + wid, 1)])
    pltpu.sync_copy(mbuf.at[pl.ds(3, 1)],
                    meta_out.at[pl.ds(3 * N_WORKERS + wid, 1)])


_sc_pass = pl.kernel(
    _sc_body,
    out_type=[
        jax.ShapeDtypeStruct((2 * N_WORKERS, D), jnp.float32),
        jax.ShapeDtypeStruct((4 * N_WORKERS, 16), jnp.float32),
    ],
    mesh=plsc.VectorSubcoreMesh(core_axis_name="c", subcore_axis_name="s"),
    scratch_types=[
        pltpu.VMEM((ROWS_PER_W + 16,), jnp.int32),
        pltpu.VMEM((2, CHUNK, D), jnp.float32),
        pltpu.VMEM((2, D), jnp.float32),
        pltpu.VMEM((4, 16), jnp.float32),
        pltpu.SemaphoreType.DMA,
        pltpu.SemaphoreType.DMA,
    ],
)


def _tc_combine_body(edge_ref, meta_ref, out_ref):
    ne = 2 * N_WORKERS
    e = edge_ref[...]
    m = meta_ref[...]
    lbl = m[:ne, 0:1]
    cnt = m[:ne, 1:2]
    sx = jnp.sum(m[ne:ne + N_WORKERS, :])
    interior = jnp.sum(m[ne + N_WORKERS:, :])
    lbl_row = lbl.reshape(1, ne)
    same = (lbl == lbl_row).astype(jnp.float32)
    gsum = jnp.dot(same, e, preferred_element_type=jnp.float32)
    gcnt = jnp.dot(same, cnt, preferred_element_type=jnp.float32)
    ir = lax.broadcasted_iota(jnp.int32, (ne, ne), 0)
    ic = lax.broadcasted_iota(jnp.int32, (ne, ne), 1)
    before = jnp.sum(same * (ic < ir).astype(jnp.float32), axis=1,
                     keepdims=True)
    first = (before == 0.0).astype(jnp.float32)
    gn2 = jnp.sum(gsum * gsum, axis=1, keepdims=True)
    contrib = jnp.sum(first * gn2 / jnp.maximum(gcnt, 1.0))
    out_ref[0, 0] = (sx - (interior + contrib)) / jnp.float32(N_ROWS * D)


_tc_combine = pl.pallas_call(
    _tc_combine_body,
    out_shape=jax.ShapeDtypeStruct((1, 1), jnp.float32),
    out_specs=pl.BlockSpec(memory_space=pltpu.SMEM),
)


def kernel(s_feature, s_labels):
    labels = s_labels.astype(jnp.int32)
    edge, meta = _sc_pass(s_feature, labels)
    out = _tc_combine(edge, meta)
    return out[0, 0]
